# baseline (device time: 26412 ns/iter reference)
import jax
import jax.numpy as jnp
from jax import lax
from jax.experimental import pallas as pl
from jax.experimental.pallas import tpu as pltpu

N_DEV = 4
BQ = 2
HG = 4
SQ = 256
DH = 64
DM = 512
DQ = 256

_MESH = pl.DeviceIdType.MESH


def kernel(x, Wq, K_ext, V_ext, Wo):
    kT = jnp.transpose(K_ext, (0, 2, 3, 1))
    vT = jnp.transpose(V_ext, (0, 2, 3, 1))

    def body(x_ref, wq_ref, kt_ref, vt_ref, wo_ref, out_ref,
             wq_all, wo_all, kts, vts,
             ssem, rsem, ksem, vsem):
        my = lax.axis_index("i")
        left = lax.rem(my + N_DEV - 1, N_DEV)
        right = lax.rem(my + 1, N_DEV)
        b0 = my * BQ

        bar = pltpu.get_barrier_semaphore()
        for nbr in (left, right):
            pl.semaphore_signal(bar, inc=1, device_id=(nbr,),
                                device_id_type=_MESH)
        pl.semaphore_wait(bar, 2)

        slot_g = [my, left, right, lax.rem(my + 2, N_DEV)]

        kv_waits = []
        for s in range(N_DEV):
            g4 = slot_g[s] * HG
            group = []
            for b in range(BQ):
                ck = pltpu.make_async_copy(
                    kt_ref.at[b0 + b, pl.ds(g4, HG)],
                    kts.at[s, b], ksem.at[s, b])
                cv = pltpu.make_async_copy(
                    vt_ref.at[b0 + b, pl.ds(g4, HG)],
                    vts.at[s, b], vsem.at[s, b])
                ck.start()
                cv.start()
                group.append((ck, cv))
            kv_waits.append(group)

        wq_all[0] = wq_ref[...].astype(jnp.bfloat16)
        wo_all[0] = wo_ref[...].astype(jnp.bfloat16)

        def rcopy(i, src, dst, dev):
            return pltpu.make_async_remote_copy(
                src_ref=src, dst_ref=dst, send_sem=ssem.at[i],
                recv_sem=rsem.at[i], device_id=(dev,), device_id_type=_MESH)

        a_wq = rcopy(0, wq_all.at[0], wq_all.at[2], left)
        a_wo = rcopy(1, wo_all.at[0], wo_all.at[2], left)
        b_wq = rcopy(2, wq_all.at[0], wq_all.at[1], right)
        b_wo = rcopy(3, wo_all.at[0], wo_all.at[1], right)
        c_wq = rcopy(4, wq_all.at[2, pl.ds(0, DM // 2)],
                     wq_all.at[3, pl.ds(0, DM // 2)], left)
        c_wo = rcopy(5, wo_all.at[2, pl.ds(0, DQ // 2)],
                     wo_all.at[3, pl.ds(0, DQ // 2)], left)
        d_wq = rcopy(6, wq_all.at[1, pl.ds(DM // 2, DM // 2)],
                     wq_all.at[3, pl.ds(DM // 2, DM // 2)], right)
        d_wo = rcopy(7, wo_all.at[1, pl.ds(DQ // 2, DQ // 2)],
                     wo_all.at[3, pl.ds(DQ // 2, DQ // 2)], right)

        qb = lax.broadcasted_iota(jnp.int32, (SQ, SQ), 0) // 64
        kb = lax.broadcasted_iota(jnp.int32, (SQ, SQ), 1) // 64
        mask = (qb == kb) | ((kb % 4) == (qb % 4))
        bias = jnp.where(mask, 0.0, -1e9).astype(jnp.float32)

        xall = (jnp.concatenate([x_ref[b] for b in range(BQ)], axis=0)
                * 0.125).astype(jnp.bfloat16)
        acc = [None]

        def attn_slot(s):
            for ck, cv in kv_waits[s]:
                ck.wait()
                cv.wait()
            wqh = wq_all[s]
            qall = jnp.dot(xall, wqh,
                           preferred_element_type=jnp.float32)
            ctxs = []
            for b in range(BQ):
                q = qall[b * SQ:(b + 1) * SQ, :]
                ctx = []
                for hh in range(HG):
                    qh = q[:, hh * DH:(hh + 1) * DH].astype(jnp.bfloat16)
                    kth = kts[s, b, hh].astype(jnp.bfloat16)
                    sc = jnp.dot(qh, kth,
                                 preferred_element_type=jnp.float32) + bias
                    e = jnp.exp(sc)
                    w = (e / jnp.sum(e, axis=1, keepdims=True)
                         ).astype(jnp.bfloat16)
                    vth = vts[s, b, hh].astype(jnp.bfloat16)
                    ctx.append(lax.dot_general(
                        w, vth, (((1,), (1,)), ((), ())),
                        preferred_element_type=jnp.float32))
                ctxs.append(jnp.concatenate(ctx, axis=1))
            return jnp.concatenate(ctxs, axis=0).astype(jnp.bfloat16)

        def out_slot(s, ctxall):
            contrib = jnp.dot(ctxall, wo_all[s],
                              preferred_element_type=jnp.float32)
            acc[0] = contrib if s == 0 else acc[0] + contrib

        for r in (a_wq, a_wo, b_wq, b_wo):
            r.start()
        out_slot(0, attn_slot(0))
        a_wq.wait()
        b_wq.wait()
        c_wq.start()
        d_wq.start()
        ctx1 = attn_slot(1)
        a_wo.wait()
        b_wo.wait()
        c_wo.start()
        d_wo.start()
        out_slot(1, ctx1)
        out_slot(2, attn_slot(2))
        c_wq.wait()
        d_wq.wait()
        ctx3 = attn_slot(3)
        c_wo.wait()
        d_wo.wait()
        out_slot(3, ctx3)
        for b in range(BQ):
            out_ref[b] = acc[0][b * SQ:(b + 1) * SQ, :].astype(jnp.bfloat16)

    out_shape = jax.ShapeDtypeStruct((BQ, SQ, DM), jnp.bfloat16)
    return pl.pallas_call(
        body,
        out_shape=out_shape,
        in_specs=[
            pl.BlockSpec(memory_space=pltpu.VMEM),
            pl.BlockSpec(memory_space=pltpu.VMEM),
            pl.BlockSpec(memory_space=pl.ANY),
            pl.BlockSpec(memory_space=pl.ANY),
            pl.BlockSpec(memory_space=pltpu.VMEM),
        ],
        out_specs=pl.BlockSpec(memory_space=pltpu.VMEM),
        scratch_shapes=[
            pltpu.VMEM((N_DEV, DM, DQ), jnp.bfloat16),
            pltpu.VMEM((N_DEV, DQ, DM), jnp.bfloat16),
            pltpu.VMEM((N_DEV, BQ, HG, DH, SQ), jnp.float32),
            pltpu.VMEM((N_DEV, BQ, HG, DH, SQ), jnp.float32),
            pltpu.SemaphoreType.DMA((8,)),
            pltpu.SemaphoreType.DMA((8,)),
            pltpu.SemaphoreType.DMA((N_DEV, BQ)),
            pltpu.SemaphoreType.DMA((N_DEV, BQ)),
        ],
        compiler_params=pltpu.CompilerParams(collective_id=0),
    )(x, Wq, kT, vT, Wo)


# device time: 18486 ns/iter; 1.4288x vs baseline; 1.4288x over previous
import jax
import jax.numpy as jnp
from jax import lax
from jax.experimental import pallas as pl
from jax.experimental.pallas import tpu as pltpu

N_DEV = 4
BQ = 2
HG = 4
SQ = 256
DH = 64
DM = 512
DQ = 256

_MESH = pl.DeviceIdType.MESH


def kernel(x, Wq, K_ext, V_ext, Wo):
    b0 = lax.axis_index("i") * BQ
    kT = jnp.transpose(
        lax.dynamic_slice_in_dim(K_ext, b0, BQ, axis=0),
        (0, 2, 3, 1))
    vT = jnp.transpose(
        lax.dynamic_slice_in_dim(V_ext, b0, BQ, axis=0),
        (0, 2, 3, 1))

    def body(x_ref, wq_ref, kt_ref, vt_ref, wo_ref, out_ref,
             wq_all, wo_all, kts, vts,
             ssem, rsem, ksem, vsem):
        my = lax.axis_index("i")
        left = lax.rem(my + N_DEV - 1, N_DEV)
        right = lax.rem(my + 1, N_DEV)

        bar = pltpu.get_barrier_semaphore()
        for nbr in (left, right):
            pl.semaphore_signal(bar, inc=1, device_id=(nbr,),
                                device_id_type=_MESH)
        pl.semaphore_wait(bar, 2)

        slot_g = [my, left, right, lax.rem(my + 2, N_DEV)]

        kv_waits = []
        for s in range(N_DEV):
            g4 = slot_g[s] * HG
            group = []
            for b in range(BQ):
                ck = pltpu.make_async_copy(
                    kt_ref.at[b, pl.ds(g4, HG)],
                    kts.at[s, b], ksem.at[s, b])
                cv = pltpu.make_async_copy(
                    vt_ref.at[b, pl.ds(g4, HG)],
                    vts.at[s, b], vsem.at[s, b])
                ck.start()
                cv.start()
                group.append((ck, cv))
            kv_waits.append(group)

        wq_all[0] = wq_ref[...].astype(jnp.bfloat16)

        def rcopy(i, src, dst, dev):
            return pltpu.make_async_remote_copy(
                src_ref=src, dst_ref=dst, send_sem=ssem.at[i],
                recv_sem=rsem.at[i], device_id=(dev,), device_id_type=_MESH)

        a_wq = rcopy(0, wq_all.at[0], wq_all.at[2], left)
        a_wo = rcopy(1, wo_all.at[0], wo_all.at[2], left)
        b_wq = rcopy(2, wq_all.at[0], wq_all.at[1], right)
        b_wo = rcopy(3, wo_all.at[0], wo_all.at[1], right)
        c_wq = rcopy(4, wq_all.at[2, pl.ds(0, DM // 2)],
                     wq_all.at[3, pl.ds(0, DM // 2)], left)
        c_wo = rcopy(5, wo_all.at[2, pl.ds(0, DQ // 2)],
                     wo_all.at[3, pl.ds(0, DQ // 2)], left)
        d_wq = rcopy(6, wq_all.at[1, pl.ds(DM // 2, DM // 2)],
                     wq_all.at[3, pl.ds(DM // 2, DM // 2)], right)
        d_wo = rcopy(7, wo_all.at[1, pl.ds(DQ // 2, DQ // 2)],
                     wo_all.at[3, pl.ds(DQ // 2, DQ // 2)], right)

        qb = lax.broadcasted_iota(jnp.int32, (SQ, SQ), 0) // 64
        kb = lax.broadcasted_iota(jnp.int32, (SQ, SQ), 1) // 64
        mask = (qb == kb) | ((kb % 4) == (qb % 4))
        bias = jnp.where(mask, 0.0, -1e9).astype(jnp.float32)

        xall = (jnp.concatenate([x_ref[b] for b in range(BQ)], axis=0)
                * 0.125).astype(jnp.bfloat16)
        acc = [None]

        def attn_slot(s):
            for ck, cv in kv_waits[s]:
                ck.wait()
                cv.wait()
            wqh = wq_all[s]
            qall = jnp.dot(xall, wqh,
                           preferred_element_type=jnp.float32)
            ctxs = []
            for b in range(BQ):
                q = qall[b * SQ:(b + 1) * SQ, :]
                ctx = []
                for hh in range(HG):
                    qh = q[:, hh * DH:(hh + 1) * DH].astype(jnp.bfloat16)
                    kth = kts[s, b, hh].astype(jnp.bfloat16)
                    sc = jnp.dot(qh, kth,
                                 preferred_element_type=jnp.float32) + bias
                    e = jnp.exp(sc)
                    w = e.astype(jnp.bfloat16)
                    vth = vts[s, b, hh].astype(jnp.bfloat16)
                    cu = lax.dot_general(
                        w, vth, (((1,), (1,)), ((), ())),
                        preferred_element_type=jnp.float32)
                    ctx.append(cu / jnp.sum(e, axis=1, keepdims=True))
                ctxs.append(jnp.concatenate(ctx, axis=1))
            return jnp.concatenate(ctxs, axis=0).astype(jnp.bfloat16)

        def out_slot(s, ctxall):
            contrib = jnp.dot(ctxall, wo_all[s],
                              preferred_element_type=jnp.float32)
            acc[0] = contrib if s == 0 else acc[0] + contrib

        a_wq.start()
        b_wq.start()
        wo_all[0] = wo_ref[...].astype(jnp.bfloat16)
        a_wo.start()
        b_wo.start()
        out_slot(0, attn_slot(0))
        a_wq.wait()
        b_wq.wait()
        c_wq.start()
        d_wq.start()
        ctx1 = attn_slot(1)
        a_wo.wait()
        b_wo.wait()
        c_wo.start()
        d_wo.start()
        out_slot(1, ctx1)
        out_slot(2, attn_slot(2))
        c_wq.wait()
        d_wq.wait()
        ctx3 = attn_slot(3)
        c_wo.wait()
        d_wo.wait()
        out_slot(3, ctx3)
        for b in range(BQ):
            out_ref[b] = acc[0][b * SQ:(b + 1) * SQ, :].astype(jnp.bfloat16)

    out_shape = jax.ShapeDtypeStruct((BQ, SQ, DM), jnp.bfloat16)
    return pl.pallas_call(
        body,
        out_shape=out_shape,
        in_specs=[
            pl.BlockSpec(memory_space=pltpu.VMEM),
            pl.BlockSpec(memory_space=pltpu.VMEM),
            pl.BlockSpec(memory_space=pl.ANY),
            pl.BlockSpec(memory_space=pl.ANY),
            pl.BlockSpec(memory_space=pltpu.VMEM),
        ],
        out_specs=pl.BlockSpec(memory_space=pltpu.VMEM),
        scratch_shapes=[
            pltpu.VMEM((N_DEV, DM, DQ), jnp.bfloat16),
            pltpu.VMEM((N_DEV, DQ, DM), jnp.bfloat16),
            pltpu.VMEM((N_DEV, BQ, HG, DH, SQ), jnp.float32),
            pltpu.VMEM((N_DEV, BQ, HG, DH, SQ), jnp.float32),
            pltpu.SemaphoreType.DMA((8,)),
            pltpu.SemaphoreType.DMA((8,)),
            pltpu.SemaphoreType.DMA((N_DEV, BQ)),
            pltpu.SemaphoreType.DMA((N_DEV, BQ)),
        ],
        compiler_params=pltpu.CompilerParams(collective_id=0),
    )(x, Wq, kT, vT, Wo)
